# baseline (speedup 1.0000x reference)
import functools

import jax
import jax.numpy as jnp
from jax import lax
from jax.experimental import pallas as pl
from jax.experimental.pallas import tpu as pltpu
from jax.experimental.pallas import tpu_sc as plsc

B = 16384
D = 32
L = 16
NT = 16
CT = B // NT
HC = D // 2
N = 1000000
HB = 524288
NB = N - HB
MARGIN = 1.0
EPS = 1e-6

_mesh = plsc.VectorSubcoreMesh(core_axis_name="c", subcore_axis_name="s")


def _sc_body(ent_t, rel_t, idx_hbm, out_hbm,
             iph, ipt, ipr, inh, int_, inr,
             al, ah, bl, bh, cl, ch, dl, dh,
             wp_h, wp_t, wp_r, wn_h, wn_t, wn_r,
             ja_h, ja_t, ja_r, jb_h, jb_t, jb_r,
             ka_h, ka_t, ka_r, kb_h, kb_t, kb_r,
             up, un, accp, accn,
             shA, shB, semA, semB, gsem):
    cid = lax.axis_index("c")
    sid = lax.axis_index("s")
    rbase = sid * CT
    c0 = cid * HC

    ivs = (iph, ipt, ipr, inh, int_, inr)
    jas = (ja_h, ja_t, ja_r, jb_h, jb_t, jb_r)
    kas = (ka_h, ka_t, ka_r, kb_h, kb_t, kb_r)
    ws = (wp_h, wp_t, wp_r, wn_h, wn_t, wn_r)
    for seg in range(6):
        pltpu.sync_copy(idx_hbm.at[pl.ds(seg * B + rbase, CT)], ivs[seg])

    def vloop(body):
        def it(j, carry):
            body(pl.ds(j * L, L))
            return carry
        lax.fori_loop(0, CT // L, it, 0)

    def prep(ds):
        for seg in range(6):
            iv = ivs[seg][ds]
            jas[seg][ds] = jnp.minimum(iv, HB - 1)
            kas[seg][ds] = jnp.clip(iv - HB, 0, NB - 1)
            ws[seg][ds] = jnp.clip(iv - (HB - 1), 0, 1).astype(jnp.float32)
    vloop(prep)

    def startA(tab, cc):
        pltpu.make_async_copy(tab.at[c0 + cc, pl.ds(0, HB)], shA, semA).start()

    def startB(tab, cc):
        pltpu.make_async_copy(tab.at[c0 + cc, pl.ds(HB, NB)], shB, semB).start()

    def waitA(tab):
        pltpu.make_async_copy(tab.at[0, pl.ds(0, HB)], shA, semA).wait()

    def waitB(tab):
        pltpu.make_async_copy(tab.at[0, pl.ds(HB, NB)], shB, semB).wait()

    def stream_phase(tab, segs, lo_bufs, hi_bufs, consume):
        def cc_body(cc, carry):
            @pl.when(sid == 0)
            def _():
                waitA(tab)
            plsc.subcore_barrier()

            @pl.when(sid == 0)
            def _():
                startB(tab, cc)

            cps = [pltpu.async_copy(shA.at[jas[s]], lo_bufs[i], gsem)
                   for i, s in enumerate(segs)]
            for cp in cps:
                cp.wait()
            plsc.subcore_barrier()

            @pl.when(sid == 0)
            def _():
                waitB(tab)
            plsc.subcore_barrier()

            @pl.when(jnp.logical_and(sid == 0, cc + 1 < HC))
            def _():
                startA(tab, cc + 1)

            cps = [pltpu.async_copy(shB.at[kas[s]], hi_bufs[i], gsem)
                   for i, s in enumerate(segs)]
            for cp in cps:
                cp.wait()
            consume(cc)
            plsc.subcore_barrier()
            return carry
        lax.fori_loop(0, HC, cc_body, 0)

    @pl.when(sid == 0)
    def _():
        startA(ent_t, 0)

    def consume_e(cc):
        def cbody(ds):
            h_p = al[ds] + wp_h[ds] * (ah[ds] - al[ds])
            t_p = bl[ds] + wp_t[ds] * (bh[ds] - bl[ds])
            h_n = cl[ds] + wn_h[ds] * (ch[ds] - cl[ds])
            t_n = dl[ds] + wn_t[ds] * (dh[ds] - dl[ds])
            up[cc, ds] = h_p - t_p
            un[cc, ds] = h_n - t_n
        vloop(cbody)

    stream_phase(ent_t, (0, 1, 3, 4), (al, bl, cl, dl), (ah, bh, ch, dh),
                 consume_e)

    def zbody(ds):
        accp[ds] = jnp.zeros((L,), jnp.float32)
        accn[ds] = jnp.zeros((L,), jnp.float32)
    vloop(zbody)

    @pl.when(sid == 0)
    def _():
        startA(rel_t, 0)

    def consume_r(cc):
        def cbody(ds):
            r_p = al[ds] + wp_r[ds] * (ah[ds] - al[ds])
            r_n = cl[ds] + wn_r[ds] * (ch[ds] - cl[ds])
            dp = up[cc, ds] + r_p + EPS
            dn = un[cc, ds] + r_n + EPS
            accp[ds] = accp[ds] + dp * dp
            accn[ds] = accn[ds] + dn * dn
        vloop(cbody)

    stream_phase(rel_t, (2, 5), (al, cl), (ah, ch), consume_r)

    pltpu.sync_copy(accp, out_hbm.at[pl.ds(cid * 2 * B + rbase, CT)])
    pltpu.sync_copy(accn, out_hbm.at[pl.ds(cid * 2 * B + B + rbase, CT)])


_sc_partials = functools.partial(
    pl.kernel,
    out_type=jax.ShapeDtypeStruct((4 * B,), jnp.float32),
    mesh=_mesh,
    scratch_types=(
        [pltpu.VMEM((CT,), jnp.int32) for _ in range(6)]
        + [pltpu.VMEM((CT,), jnp.float32) for _ in range(8)]
        + [pltpu.VMEM((CT,), jnp.float32) for _ in range(6)]
        + [pltpu.VMEM((CT,), jnp.int32) for _ in range(12)]
        + [pltpu.VMEM((HC, CT), jnp.float32),
           pltpu.VMEM((HC, CT), jnp.float32),
           pltpu.VMEM((CT,), jnp.float32),
           pltpu.VMEM((CT,), jnp.float32),
           pltpu.VMEM_SHARED((HB,), jnp.float32),
           pltpu.VMEM_SHARED((NB,), jnp.float32),
           pltpu.SemaphoreType.DMA,
           pltpu.SemaphoreType.DMA,
           pltpu.SemaphoreType.DMA]
    ),
)(_sc_body)


def _finish_body(x_ref, o_ref):
    x = x_ref[...]
    d2p = x[0:1] + x[2:3]
    d2n = x[1:2] + x[3:4]
    m = jnp.sqrt(d2p) - jnp.sqrt(d2n) + MARGIN
    o_ref[...] = jnp.sum(jnp.maximum(m, 0.0), keepdims=True) * (1.0 / B)


_finish = pl.pallas_call(
    _finish_body,
    out_shape=jax.ShapeDtypeStruct((1, 1), jnp.float32),
)


def kernel(pos_x, neg_x, entity_weight, relation_weight):
    pos = pos_x.astype(jnp.int32)
    neg = neg_x.astype(jnp.int32)
    idx_flat = jnp.concatenate([
        pos[:, 0], pos[:, 2], pos[:, 1],
        neg[:, 0], neg[:, 2], neg[:, 1],
    ])
    part = _sc_partials(entity_weight.T, relation_weight.T, idx_flat)
    return _finish(part.reshape(4, B))[0, 0]


# speedup vs baseline: 1.5122x; 1.5122x over previous
import functools

import jax
import jax.numpy as jnp
from jax import lax
from jax.experimental import pallas as pl
from jax.experimental.pallas import tpu as pltpu
from jax.experimental.pallas import tpu_sc as plsc

B = 16384
D = 32
L = 16
NW = 32
C = B // NW
TCH = 128
NCH = C // TCH
N = 1000000
NMAIN = 999936
GMAIN = NMAIN // 4
MARGIN = 1.0
EPS = 1e-6

_mesh = plsc.VectorSubcoreMesh(core_axis_name="c", subcore_axis_name="s")


def _sc_body(ent_hbm, rel_hbm, etail_hbm, idx_hbm, out_hbm,
             iph, ipt, ipr, inh, int_, inr,
             gph, gpt, gpr, gnh, gnt, gnr,
             bph, bpt, bpr, bnh, bnt, bnr,
             etv, sqp, sqn, sem):
    cid = lax.axis_index("c")
    sid = lax.axis_index("s")
    wid = sid * 2 + cid
    base = wid * C

    ivs = (iph, ipt, ipr, inh, int_, inr)
    gvs = (gph, gpt, gpr, gnh, gnt, gnr)
    bufs = (bph, bpt, bpr, bnh, bnt, bnr)
    tabs = (ent_hbm, ent_hbm, rel_hbm, ent_hbm, ent_hbm, rel_hbm)

    pltpu.sync_copy(etail_hbm, etv)

    for seg in range(6):
        pltpu.sync_copy(idx_hbm.at[pl.ds(seg * B + base, C)], ivs[seg])

    def shift_body(j, carry):
        for seg in range(6):
            iv = ivs[seg][pl.ds(j * L, L)]
            if seg in (2, 5):
                gvs[seg][pl.ds(j * L, L)] = iv >> 2
            else:
                g = ((iv >> 9) << 7) | (iv & 127)
                gvs[seg][pl.ds(j * L, L)] = jnp.minimum(g, GMAIN - 1)
        return carry
    lax.fori_loop(0, C // L, shift_body, 0)

    def pick_rel(buf, iv_chunk, k, r):
        idx_s = iv_chunk[k]
        q32 = (idx_s & 3) * D
        return buf[r, pl.ds(q32, L)], buf[r, pl.ds(q32 + L, L)]

    def pick(buf, tv, iv_chunk, k, r):
        idx_s = iv_chunk[k]
        q32m = ((idx_s >> 7) & 3) * D
        e_t = idx_s - NMAIN
        tr = jnp.clip(e_t >> 2, 0, 15)
        q32t = (e_t & 3) * D
        w = jnp.where(idx_s >= NMAIN, 1.0, 0.0)
        m0 = buf[r, pl.ds(q32m, L)]
        m1 = buf[r, pl.ds(q32m + L, L)]
        t0 = tv[tr, pl.ds(q32t, L)]
        t1 = tv[tr, pl.ds(q32t + L, L)]
        return m0 + w * (t0 - m0), m1 + w * (t1 - m1)

    def chunk_body(ch, carry):
        cbase = ch * TCH
        cps = [
            pltpu.async_copy(tabs[seg].at[gvs[seg].at[pl.ds(cbase, TCH)]],
                             bufs[seg], sem)
            for seg in range(6)
        ]
        for cp in cps:
            cp.wait()

        def group_body(g, carry2):
            gr0 = cbase + g * L
            ivc = [ivs[seg][pl.ds(gr0, L)] for seg in range(6)]
            for k in range(L):
                r = g * L + k
                h0, h1 = pick(bph, etv, ivc[0], k, r)
                t0, t1 = pick(bpt, etv, ivc[1], k, r)
                r0, r1 = pick_rel(bpr, ivc[2], k, r)
                d0 = h0 + r0 - t0 + EPS
                d1 = h1 + r1 - t1 + EPS
                sqp[pl.ds((gr0 + k) * L, L)] = d0 * d0 + d1 * d1
                h0, h1 = pick(bnh, etv, ivc[3], k, r)
                t0, t1 = pick(bnt, etv, ivc[4], k, r)
                r0, r1 = pick_rel(bnr, ivc[5], k, r)
                d0 = h0 + r0 - t0 + EPS
                d1 = h1 + r1 - t1 + EPS
                sqn[pl.ds((gr0 + k) * L, L)] = d0 * d0 + d1 * d1
            return carry2

        lax.fori_loop(0, TCH // L, group_body, 0)
        return carry

    lax.fori_loop(0, NCH, chunk_body, 0)

    pltpu.sync_copy(sqp, out_hbm.at[pl.ds(base * L, C * L)])
    pltpu.sync_copy(sqn, out_hbm.at[pl.ds(B * L + base * L, C * L)])


_sc_distances = functools.partial(
    pl.kernel,
    out_type=jax.ShapeDtypeStruct((2 * B * L,), jnp.float32),
    mesh=_mesh,
    scratch_types=(
        [pltpu.VMEM((C,), jnp.int32) for _ in range(6)]
        + [pltpu.VMEM((C,), jnp.int32) for _ in range(6)]
        + [pltpu.VMEM((TCH, 128), jnp.float32) for _ in range(6)]
        + [pltpu.VMEM((16, 128), jnp.float32),
           pltpu.VMEM((C * L,), jnp.float32),
           pltpu.VMEM((C * L,), jnp.float32),
           pltpu.SemaphoreType.DMA]
    ),
)(_sc_body)

_W = 15872
_NBLK = NMAIN // _W


def _t_body(x_ref, o_ref):
    y = jnp.transpose(x_ref[...])
    y4 = y.reshape(_W // 512, 4, 128, D)
    for q in range(4):
        o_ref[:, q * D:(q + 1) * D] = y4[:, q].reshape(_W // 4, D)


_to_rows = pl.pallas_call(
    _t_body,
    grid=(_NBLK,),
    in_specs=[pl.BlockSpec((32, _W), lambda j: (0, j))],
    out_specs=pl.BlockSpec((_W // 4, 128), lambda j: (j, 0)),
    out_shape=jax.ShapeDtypeStruct((GMAIN, 128), jnp.float32),
)

_ROWS = 2 * B * L // 128
_HALF = _ROWS // 2


def _finish_body(x_ref, o_ref):
    x = x_ref[...]
    i128 = lax.broadcasted_iota(jnp.int32, (128, 8), 0)
    i8 = lax.broadcasted_iota(jnp.int32, (128, 8), 1)
    s_mat = jnp.where(i128 // L == i8, 1.0, 0.0).astype(jnp.float32)
    d2p = jnp.dot(x[:_HALF], s_mat, preferred_element_type=jnp.float32)
    d2n = jnp.dot(x[_HALF:], s_mat, preferred_element_type=jnp.float32)
    m = jnp.sqrt(d2p) - jnp.sqrt(d2n) + MARGIN
    o_ref[...] = jnp.sum(jnp.maximum(m, 0.0), keepdims=True) * (1.0 / B)


_finish = pl.pallas_call(
    _finish_body,
    out_shape=jax.ShapeDtypeStruct((1, 1), jnp.float32),
)


def kernel(pos_x, neg_x, entity_weight, relation_weight):
    pos = pos_x.astype(jnp.int32)
    neg = neg_x.astype(jnp.int32)
    idx_flat = jnp.concatenate([
        pos[:, 0], pos[:, 2], pos[:, 1],
        neg[:, 0], neg[:, 2], neg[:, 1],
    ])
    ent4 = _to_rows(entity_weight.T)
    rel4 = relation_weight.reshape(-1, 128)
    etail = entity_weight[NMAIN:].reshape(16, 128)
    sq = _sc_distances(ent4, rel4, etail, idx_flat)
    return _finish(sq.reshape(_ROWS, 128))[0, 0]


# speedup vs baseline: 1.8166x; 1.2013x over previous
import functools

import jax
import jax.numpy as jnp
from jax import lax
from jax.experimental import pallas as pl
from jax.experimental.pallas import tpu as pltpu
from jax.experimental.pallas import tpu_sc as plsc

B = 16384
D = 32
L = 16
NW = 32
C = B // NW
TCH = 128
NCH = C // TCH
N = 1000000
NMAIN = 999936
GMAIN = NMAIN // 4
MARGIN = 1.0
EPS = 1e-6

_mesh = plsc.VectorSubcoreMesh(core_axis_name="c", subcore_axis_name="s")


def _gidx_map(iv):
    g = ((iv >> 9) << 7) | (iv & 127)
    return jnp.minimum(g, GMAIN - 1)


def _pick(buf, tv, iv_chunk, k, r):
    idx_s = iv_chunk[k]
    q32m = ((idx_s >> 7) & 3) * D
    e_t = idx_s - NMAIN
    tr = jnp.clip(e_t >> 2, 0, 15)
    q32t = (e_t & 3) * D
    w = jnp.where(idx_s >= NMAIN, 1.0, 0.0)
    m0 = buf[r, pl.ds(q32m, L)]
    m1 = buf[r, pl.ds(q32m + L, L)]
    t0 = tv[tr, pl.ds(q32t, L)]
    t1 = tv[tr, pl.ds(q32t + L, L)]
    return m0 + w * (t0 - m0), m1 + w * (t1 - m1)


def _u_body(ent_hbm, etail_hbm, idx_hbm, uout_hbm,
            iph, ipt, inh, int_,
            gph, gpt, gnh, gnt,
            bph, bpt, bnh, bnt,
            etv, upv, unv, sem):
    cid = lax.axis_index("c")
    sid = lax.axis_index("s")
    base = (sid * 2 + cid) * C

    ivs = (iph, ipt, inh, int_)
    gvs = (gph, gpt, gnh, gnt)
    bufs = (bph, bpt, bnh, bnt)
    segs = (0, 1, 3, 4)

    pltpu.sync_copy(etail_hbm, etv)
    for i, seg in enumerate(segs):
        pltpu.sync_copy(idx_hbm.at[pl.ds(seg * B + base, C)], ivs[i])

    def shift_body(j, carry):
        for i in range(4):
            gvs[i][pl.ds(j * L, L)] = _gidx_map(ivs[i][pl.ds(j * L, L)])
        return carry
    lax.fori_loop(0, C // L, shift_body, 0)

    def chunk_body(ch, carry):
        cbase = ch * TCH
        cps = [
            pltpu.async_copy(ent_hbm.at[gvs[i].at[pl.ds(cbase, TCH)]],
                             bufs[i], sem)
            for i in range(4)
        ]
        for cp in cps:
            cp.wait()

        def group_body(g, carry2):
            gr0 = cbase + g * L
            ivc = [ivs[i][pl.ds(gr0, L)] for i in range(4)]
            for k in range(L):
                r = g * L + k
                h0, h1 = _pick(bph, etv, ivc[0], k, r)
                t0, t1 = _pick(bpt, etv, ivc[1], k, r)
                upv[pl.ds((gr0 + k) * D, L)] = h0 - t0
                upv[pl.ds((gr0 + k) * D + L, L)] = h1 - t1
                h0, h1 = _pick(bnh, etv, ivc[2], k, r)
                t0, t1 = _pick(bnt, etv, ivc[3], k, r)
                unv[pl.ds((gr0 + k) * D, L)] = h0 - t0
                unv[pl.ds((gr0 + k) * D + L, L)] = h1 - t1
            return carry2

        lax.fori_loop(0, TCH // L, group_body, 0)
        return carry

    lax.fori_loop(0, NCH, chunk_body, 0)

    pltpu.sync_copy(upv, uout_hbm.at[pl.ds(base * D, C * D)])
    pltpu.sync_copy(unv, uout_hbm.at[pl.ds(B * D + base * D, C * D)])


_sc_u = functools.partial(
    pl.kernel,
    out_type=jax.ShapeDtypeStruct((2 * B * D,), jnp.float32),
    mesh=_mesh,
    scratch_types=(
        [pltpu.VMEM((C,), jnp.int32) for _ in range(4)]
        + [pltpu.VMEM((C,), jnp.int32) for _ in range(4)]
        + [pltpu.VMEM((TCH, 128), jnp.float32) for _ in range(4)]
        + [pltpu.VMEM((16, 128), jnp.float32),
           pltpu.VMEM((C * D,), jnp.float32),
           pltpu.VMEM((C * D,), jnp.float32),
           pltpu.SemaphoreType.DMA]
    ),
)(_u_body)


def _sq_body(rel_hbm, rtail_hbm, idx_hbm, u_hbm, out_hbm,
             ipr, inr, gpr, gnr, bpr, bnr,
             rtv, upv, unv, sqp, sqn, sem):
    cid = lax.axis_index("c")
    sid = lax.axis_index("s")
    base = (sid * 2 + cid) * C

    pltpu.sync_copy(rtail_hbm, rtv)
    pltpu.sync_copy(idx_hbm.at[pl.ds(2 * B + base, C)], ipr)
    pltpu.sync_copy(idx_hbm.at[pl.ds(5 * B + base, C)], inr)
    pltpu.sync_copy(u_hbm.at[pl.ds(base * D, C * D)], upv)
    pltpu.sync_copy(u_hbm.at[pl.ds(B * D + base * D, C * D)], unv)

    def shift_body(j, carry):
        gpr[pl.ds(j * L, L)] = _gidx_map(ipr[pl.ds(j * L, L)])
        gnr[pl.ds(j * L, L)] = _gidx_map(inr[pl.ds(j * L, L)])
        return carry
    lax.fori_loop(0, C // L, shift_body, 0)

    def chunk_body(ch, carry):
        cbase = ch * TCH
        cps = [
            pltpu.async_copy(rel_hbm.at[gpr.at[pl.ds(cbase, TCH)]], bpr, sem),
            pltpu.async_copy(rel_hbm.at[gnr.at[pl.ds(cbase, TCH)]], bnr, sem),
        ]
        for cp in cps:
            cp.wait()

        def group_body(g, carry2):
            gr0 = cbase + g * L
            ivcp = ipr[pl.ds(gr0, L)]
            ivcn = inr[pl.ds(gr0, L)]
            for k in range(L):
                r = g * L + k
                r0, r1 = _pick(bpr, rtv, ivcp, k, r)
                d0 = upv[pl.ds((gr0 + k) * D, L)] + r0 + EPS
                d1 = upv[pl.ds((gr0 + k) * D + L, L)] + r1 + EPS
                sqp[pl.ds((gr0 + k) * L, L)] = d0 * d0 + d1 * d1
                r0, r1 = _pick(bnr, rtv, ivcn, k, r)
                d0 = unv[pl.ds((gr0 + k) * D, L)] + r0 + EPS
                d1 = unv[pl.ds((gr0 + k) * D + L, L)] + r1 + EPS
                sqn[pl.ds((gr0 + k) * L, L)] = d0 * d0 + d1 * d1
            return carry2

        lax.fori_loop(0, TCH // L, group_body, 0)
        return carry

    lax.fori_loop(0, NCH, chunk_body, 0)

    pltpu.sync_copy(sqp, out_hbm.at[pl.ds(base * L, C * L)])
    pltpu.sync_copy(sqn, out_hbm.at[pl.ds(B * L + base * L, C * L)])


_sc_sq = functools.partial(
    pl.kernel,
    out_type=jax.ShapeDtypeStruct((2 * B * L,), jnp.float32),
    mesh=_mesh,
    scratch_types=(
        [pltpu.VMEM((C,), jnp.int32) for _ in range(2)]
        + [pltpu.VMEM((C,), jnp.int32) for _ in range(2)]
        + [pltpu.VMEM((TCH, 128), jnp.float32) for _ in range(2)]
        + [pltpu.VMEM((16, 128), jnp.float32),
           pltpu.VMEM((C * D,), jnp.float32),
           pltpu.VMEM((C * D,), jnp.float32),
           pltpu.VMEM((C * L,), jnp.float32),
           pltpu.VMEM((C * L,), jnp.float32),
           pltpu.SemaphoreType.DMA]
    ),
)(_sq_body)

_W = 15872
_NBLK = NMAIN // _W


def _t_body(x_ref, o_ref):
    y = jnp.transpose(x_ref[...])
    y4 = y.reshape(_W // 512, 4, 128, D)
    for q in range(4):
        o_ref[:, q * D:(q + 1) * D] = y4[:, q].reshape(_W // 4, D)


_to_rows = pl.pallas_call(
    _t_body,
    grid=(_NBLK,),
    in_specs=[pl.BlockSpec((32, _W), lambda j: (0, j))],
    out_specs=pl.BlockSpec((_W // 4, 128), lambda j: (j, 0)),
    out_shape=jax.ShapeDtypeStruct((GMAIN, 128), jnp.float32),
)

_ROWS = 2 * B * L // 128
_HALF = _ROWS // 2


def _finish_body(x_ref, o_ref):
    x = x_ref[...]
    i128 = lax.broadcasted_iota(jnp.int32, (128, 8), 0)
    i8 = lax.broadcasted_iota(jnp.int32, (128, 8), 1)
    s_mat = jnp.where(i128 // L == i8, 1.0, 0.0).astype(jnp.float32)
    d2p = jnp.dot(x[:_HALF], s_mat, preferred_element_type=jnp.float32)
    d2n = jnp.dot(x[_HALF:], s_mat, preferred_element_type=jnp.float32)
    m = jnp.sqrt(d2p) - jnp.sqrt(d2n) + MARGIN
    o_ref[...] = jnp.sum(jnp.maximum(m, 0.0), keepdims=True) * (1.0 / B)


_finish = pl.pallas_call(
    _finish_body,
    out_shape=jax.ShapeDtypeStruct((1, 1), jnp.float32),
)


def kernel(pos_x, neg_x, entity_weight, relation_weight):
    pos = pos_x.astype(jnp.int32)
    neg = neg_x.astype(jnp.int32)
    idx_flat = jnp.concatenate([
        pos[:, 0], pos[:, 2], pos[:, 1],
        neg[:, 0], neg[:, 2], neg[:, 1],
    ])
    ent4 = _to_rows(entity_weight.T)
    etail = entity_weight[NMAIN:].reshape(16, 128)
    u = _sc_u(ent4, etail, idx_flat)
    rel4 = _to_rows(relation_weight.T)
    rtail = relation_weight[NMAIN:].reshape(16, 128)
    sq = _sc_sq(rel4, rtail, idx_flat, u)
    return _finish(sq.reshape(_ROWS, 128))[0, 0]


# speedup vs baseline: 1.8336x; 1.0094x over previous
import functools

import jax
import jax.numpy as jnp
from jax import lax
from jax.experimental import pallas as pl
from jax.experimental.pallas import tpu as pltpu
from jax.experimental.pallas import tpu_sc as plsc

B = 16384
D = 32
L = 16
NW = 32
C = B // NW
TCH = 128
NCH = C // TCH
N = 1000000
NMAIN = 999936
GMAIN = NMAIN // 4
MARGIN = 1.0
EPS = 1e-6

_mesh = plsc.VectorSubcoreMesh(core_axis_name="c", subcore_axis_name="s")


def _gidx_map(iv):
    g = ((iv >> 9) << 7) | (iv & 127)
    return jnp.minimum(g, GMAIN - 1)


def _pick(buf, tv, iv_chunk, k, r):
    idx_s = iv_chunk[k]
    q32m = ((idx_s >> 7) & 3) * D
    e_t = idx_s - NMAIN
    tr = jnp.clip(e_t >> 2, 0, 15)
    q32t = (e_t & 3) * D
    w = jnp.where(idx_s >= NMAIN, 1.0, 0.0)
    m0 = buf[r, pl.ds(q32m, L)]
    m1 = buf[r, pl.ds(q32m + L, L)]
    t0 = tv[tr, pl.ds(q32t, L)]
    t1 = tv[tr, pl.ds(q32t + L, L)]
    return m0 + w * (t0 - m0), m1 + w * (t1 - m1)


def _u_body(ent_hbm, etail_hbm, idx_hbm, uout_hbm,
            iph, ipt, inh, int_,
            gph, gpt, gnh, gnt,
            bph, bpt, bnh, bnt,
            etv, upv, unv, sem):
    cid = lax.axis_index("c")
    sid = lax.axis_index("s")
    base = (sid * 2 + cid) * C

    ivs = (iph, ipt, inh, int_)
    gvs = (gph, gpt, gnh, gnt)
    bufs = (bph, bpt, bnh, bnt)
    segs = (0, 1, 3, 4)

    pltpu.sync_copy(etail_hbm, etv)
    for i, seg in enumerate(segs):
        pltpu.sync_copy(idx_hbm.at[pl.ds(seg * B + base, C)], ivs[i])

    def shift_body(j, carry):
        for i in range(4):
            gvs[i][pl.ds(j * L, L)] = _gidx_map(ivs[i][pl.ds(j * L, L)])
        return carry
    lax.fori_loop(0, C // L, shift_body, 0)

    def chunk_body(ch, carry):
        cbase = ch * TCH
        cps = [
            pltpu.async_copy(ent_hbm.at[gvs[i].at[pl.ds(cbase, TCH)]],
                             bufs[i], sem)
            for i in range(4)
        ]
        for cp in cps:
            cp.wait()

        def group_body(g, carry2):
            gr0 = cbase + g * L
            ivc = [ivs[i][pl.ds(gr0, L)] for i in range(4)]
            for k in range(L):
                r = g * L + k
                h0, h1 = _pick(bph, etv, ivc[0], k, r)
                t0, t1 = _pick(bpt, etv, ivc[1], k, r)
                upv[pl.ds((gr0 + k) * D, L)] = h0 - t0
                upv[pl.ds((gr0 + k) * D + L, L)] = h1 - t1
                h0, h1 = _pick(bnh, etv, ivc[2], k, r)
                t0, t1 = _pick(bnt, etv, ivc[3], k, r)
                unv[pl.ds((gr0 + k) * D, L)] = h0 - t0
                unv[pl.ds((gr0 + k) * D + L, L)] = h1 - t1
            return carry2

        lax.fori_loop(0, TCH // L, group_body, 0)
        return carry

    lax.fori_loop(0, NCH, chunk_body, 0)

    pltpu.sync_copy(upv, uout_hbm.at[pl.ds(base * D, C * D)])
    pltpu.sync_copy(unv, uout_hbm.at[pl.ds(B * D + base * D, C * D)])


_sc_u = functools.partial(
    pl.kernel,
    out_type=jax.ShapeDtypeStruct((2 * B * D,), jnp.float32),
    mesh=_mesh,
    scratch_types=(
        [pltpu.VMEM((C,), jnp.int32) for _ in range(4)]
        + [pltpu.VMEM((C,), jnp.int32) for _ in range(4)]
        + [pltpu.VMEM((TCH, 128), jnp.float32) for _ in range(4)]
        + [pltpu.VMEM((16, 128), jnp.float32),
           pltpu.VMEM((C * D,), jnp.float32),
           pltpu.VMEM((C * D,), jnp.float32),
           pltpu.SemaphoreType.DMA]
    ),
)(_u_body)


def _sq_body(rel_hbm, rtail_hbm, idx_hbm, u_hbm, out_hbm,
             ipr, inr, gpr, gnr, bpr, bnr,
             rtv, upv, unv, sqp, sqn, sem):
    cid = lax.axis_index("c")
    sid = lax.axis_index("s")
    base = (sid * 2 + cid) * C

    pltpu.sync_copy(rtail_hbm, rtv)
    pltpu.sync_copy(idx_hbm.at[pl.ds(2 * B + base, C)], ipr)
    pltpu.sync_copy(idx_hbm.at[pl.ds(5 * B + base, C)], inr)
    pltpu.sync_copy(u_hbm.at[pl.ds(base * D, C * D)], upv)
    pltpu.sync_copy(u_hbm.at[pl.ds(B * D + base * D, C * D)], unv)

    def shift_body(j, carry):
        gpr[pl.ds(j * L, L)] = _gidx_map(ipr[pl.ds(j * L, L)])
        gnr[pl.ds(j * L, L)] = _gidx_map(inr[pl.ds(j * L, L)])
        return carry
    lax.fori_loop(0, C // L, shift_body, 0)

    def chunk_body(ch, carry):
        cbase = ch * TCH
        cps = [
            pltpu.async_copy(rel_hbm.at[gpr.at[pl.ds(cbase, TCH)]], bpr, sem),
            pltpu.async_copy(rel_hbm.at[gnr.at[pl.ds(cbase, TCH)]], bnr, sem),
        ]
        for cp in cps:
            cp.wait()

        def group_body(g, carry2):
            gr0 = cbase + g * L
            ivcp = ipr[pl.ds(gr0, L)]
            ivcn = inr[pl.ds(gr0, L)]
            for k in range(L):
                r = g * L + k
                r0, r1 = _pick(bpr, rtv, ivcp, k, r)
                d0 = upv[pl.ds((gr0 + k) * D, L)] + r0 + EPS
                d1 = upv[pl.ds((gr0 + k) * D + L, L)] + r1 + EPS
                sqp[pl.ds((gr0 + k) * L, L)] = d0 * d0 + d1 * d1
                r0, r1 = _pick(bnr, rtv, ivcn, k, r)
                d0 = unv[pl.ds((gr0 + k) * D, L)] + r0 + EPS
                d1 = unv[pl.ds((gr0 + k) * D + L, L)] + r1 + EPS
                sqn[pl.ds((gr0 + k) * L, L)] = d0 * d0 + d1 * d1
            return carry2

        lax.fori_loop(0, TCH // L, group_body, 0)
        return carry

    lax.fori_loop(0, NCH, chunk_body, 0)

    pltpu.sync_copy(sqp, out_hbm.at[pl.ds(base * L, C * L)])
    pltpu.sync_copy(sqn, out_hbm.at[pl.ds(B * L + base * L, C * L)])


_sc_sq = functools.partial(
    pl.kernel,
    out_type=jax.ShapeDtypeStruct((2 * B * L,), jnp.float32),
    mesh=_mesh,
    scratch_types=(
        [pltpu.VMEM((C,), jnp.int32) for _ in range(2)]
        + [pltpu.VMEM((C,), jnp.int32) for _ in range(2)]
        + [pltpu.VMEM((TCH, 128), jnp.float32) for _ in range(2)]
        + [pltpu.VMEM((16, 128), jnp.float32),
           pltpu.VMEM((C * D,), jnp.float32),
           pltpu.VMEM((C * D,), jnp.float32),
           pltpu.VMEM((C * L,), jnp.float32),
           pltpu.VMEM((C * L,), jnp.float32),
           pltpu.SemaphoreType.DMA]
    ),
)(_sq_body)

_W = 32256
_NBLK = NMAIN // _W


def _t_body(x_ref, o_ref):
    y = jnp.transpose(x_ref[...])
    y4 = y.reshape(_W // 512, 4, 128, D)
    for q in range(4):
        o_ref[:, q * D:(q + 1) * D] = y4[:, q].reshape(_W // 4, D)


_to_rows = pl.pallas_call(
    _t_body,
    grid=(_NBLK,),
    in_specs=[pl.BlockSpec((32, _W), lambda j: (0, j))],
    out_specs=pl.BlockSpec((_W // 4, 128), lambda j: (j, 0)),
    out_shape=jax.ShapeDtypeStruct((GMAIN, 128), jnp.float32),
)

_ROWS = 2 * B * L // 128
_HALF = _ROWS // 2


def _finish_body(x_ref, o_ref):
    x = x_ref[...]
    i128 = lax.broadcasted_iota(jnp.int32, (128, 8), 0)
    i8 = lax.broadcasted_iota(jnp.int32, (128, 8), 1)
    s_mat = jnp.where(i128 // L == i8, 1.0, 0.0).astype(jnp.float32)
    d2p = jnp.dot(x[:_HALF], s_mat, preferred_element_type=jnp.float32)
    d2n = jnp.dot(x[_HALF:], s_mat, preferred_element_type=jnp.float32)
    m = jnp.sqrt(d2p) - jnp.sqrt(d2n) + MARGIN
    o_ref[...] = jnp.sum(jnp.maximum(m, 0.0), keepdims=True) * (1.0 / B)


_finish = pl.pallas_call(
    _finish_body,
    out_shape=jax.ShapeDtypeStruct((1, 1), jnp.float32),
)


def kernel(pos_x, neg_x, entity_weight, relation_weight):
    pos = pos_x.astype(jnp.int32)
    neg = neg_x.astype(jnp.int32)
    idx_flat = jnp.concatenate([
        pos[:, 0], pos[:, 2], pos[:, 1],
        neg[:, 0], neg[:, 2], neg[:, 1],
    ])
    ent4 = _to_rows(entity_weight.T)
    etail = entity_weight[NMAIN:].reshape(16, 128)
    u = _sc_u(ent4, etail, idx_flat)
    rel4 = _to_rows(relation_weight.T)
    rtail = relation_weight[NMAIN:].reshape(16, 128)
    sq = _sc_sq(rel4, rtail, idx_flat, u)
    return _finish(sq.reshape(_ROWS, 128))[0, 0]
